# baseline (speedup 1.0000x reference)
import functools

import jax
import jax.numpy as jnp
from jax.experimental import pallas as pl
from jax.experimental.pallas import tpu as pltpu

B, C, N, M = 1024, 1024, 16384, 64
BT = 32
WPAD = 128


def _read_head_body(x_ref, wpre_ref, W_ref, b_ref, memt_ref, r_ref, w_ref,
                    mnt_ref, fc_ref):
    memt = memt_ref[...]
    i = pl.program_id(0)

    @pl.when(i == 0)
    def _():
        mnorm = jnp.sqrt(jnp.sum(memt * memt, axis=0, keepdims=True))
        mnt_ref[...] = memt * (1.0 / (mnorm + 1e-8))
        fc_ref[...] = jax.lax.dot_general(
            x_ref[...], W_ref[...], (((1,), (1,)), ((), ())),
            preferred_element_type=jnp.float32) + b_ref[...]

    out = fc_ref[pl.ds(i * BT, BT), :]

    k = out[:, :M]
    beta = jax.nn.softplus(out[:, M:M + 1])
    g = jax.nn.sigmoid(out[:, M + 1:M + 2])
    s = jax.nn.softmax(out[:, M + 2:M + 5], axis=1)
    gamma = 1.0 + jax.nn.softplus(out[:, M + 5:M + 6])

    knorm = jnp.sqrt(jnp.sum(k * k, axis=1, keepdims=True))
    kn = k / (knorm + 1e-8)
    sim = jax.lax.dot_general(
        kn, mnt_ref[...], (((1,), (0,)), ((), ())),
        preferred_element_type=jnp.float32)

    e = jnp.exp2((beta * 1.4426950408889634) * sim)
    c1 = g / jnp.sum(e, axis=1, keepdims=True)
    wg = c1 * e + (1.0 - g) * wpre_ref[...]

    ws = (s[:, 0:1] * pltpu.roll(wg, N - 1, 1)
          + s[:, 1:2] * wg
          + s[:, 2:3] * pltpu.roll(wg, 1, 1))

    wp = jnp.exp2(gamma * jnp.log2(ws + 1e-12))
    w = wp * (1.0 / (jnp.sum(wp, axis=1, keepdims=True) + 1e-12))

    w_ref[...] = w
    r_ref[...] = jax.lax.dot_general(
        w, memt, (((1,), (1,)), ((), ())),
        preferred_element_type=jnp.float32)


@jax.jit
def kernel(inputs, w_pre, W, b, mem):
    Wp = jnp.zeros((WPAD, C), jnp.float32).at[:M + 6, :].set(W)
    bp = jnp.zeros((1, WPAD), jnp.float32).at[0, :M + 6].set(b)
    memt = mem.T

    grid = (B // BT,)
    r, w = pl.pallas_call(
        _read_head_body,
        grid=grid,
        in_specs=[
            pl.BlockSpec((B, C), lambda i: (0, 0)),
            pl.BlockSpec((BT, N), lambda i: (i, 0)),
            pl.BlockSpec((WPAD, C), lambda i: (0, 0)),
            pl.BlockSpec((1, WPAD), lambda i: (0, 0)),
            pl.BlockSpec((M, N), lambda i: (0, 0)),
        ],
        out_specs=[
            pl.BlockSpec((BT, M), lambda i: (i, 0)),
            pl.BlockSpec((BT, N), lambda i: (i, 0)),
        ],
        out_shape=[
            jax.ShapeDtypeStruct((B, M), jnp.float32),
            jax.ShapeDtypeStruct((B, N), jnp.float32),
        ],
        scratch_shapes=[pltpu.VMEM((M, N), jnp.float32),
                        pltpu.VMEM((B, WPAD), jnp.float32)],
        compiler_params=pltpu.CompilerParams(
            dimension_semantics=("arbitrary",),
        ),
    )(inputs, w_pre, Wp, bp, memt)
    return (r, w)


# speedup vs baseline: 1.0697x; 1.0697x over previous
import functools

import jax
import jax.numpy as jnp
from jax.experimental import pallas as pl
from jax.experimental.pallas import tpu as pltpu

B, C, N, M = 1024, 1024, 16384, 64
BT = 64
WPAD = 128


def _read_head_body(x_ref, wpre_ref, W_ref, b_ref, memt_ref, r_ref, w_ref,
                    mnt_ref, fc_ref):
    memt = memt_ref[...]
    i = pl.program_id(0)

    @pl.when(i == 0)
    def _():
        mnorm = jnp.sqrt(jnp.sum(memt * memt, axis=0, keepdims=True))
        mnt_ref[...] = memt * (1.0 / (mnorm + 1e-8))
        fc_ref[...] = jax.lax.dot_general(
            x_ref[...], W_ref[...], (((1,), (1,)), ((), ())),
            preferred_element_type=jnp.float32) + b_ref[...]

    out = fc_ref[pl.ds(i * BT, BT), :]

    k = out[:, :M]
    beta = jax.nn.softplus(out[:, M:M + 1])
    g = jax.nn.sigmoid(out[:, M + 1:M + 2])
    s = jax.nn.softmax(out[:, M + 2:M + 5], axis=1)
    gamma = 1.0 + jax.nn.softplus(out[:, M + 5:M + 6])

    knorm = jnp.sqrt(jnp.sum(k * k, axis=1, keepdims=True))
    kn = k / (knorm + 1e-8)
    sim = jax.lax.dot_general(
        kn, mnt_ref[...], (((1,), (0,)), ((), ())),
        preferred_element_type=jnp.float32)

    e = jnp.exp2((beta * 1.4426950408889634) * sim)
    c1 = g / jnp.sum(e, axis=1, keepdims=True)
    wg = c1 * e + (1.0 - g) * wpre_ref[...]

    ws = (s[:, 0:1] * jnp.roll(wg, -1, 1)
          + s[:, 1:2] * wg
          + s[:, 2:3] * jnp.roll(wg, 1, 1))

    wp = jnp.exp2(gamma * jnp.log2(ws + 1e-12))
    w = wp * (1.0 / (jnp.sum(wp, axis=1, keepdims=True) + 1e-12))

    w_ref[...] = w
    r_ref[...] = jax.lax.dot_general(
        w, memt, (((1,), (1,)), ((), ())),
        preferred_element_type=jnp.float32)


@jax.jit
def kernel(inputs, w_pre, W, b, mem):
    Wp = jnp.zeros((WPAD, C), jnp.float32).at[:M + 6, :].set(W)
    bp = jnp.zeros((1, WPAD), jnp.float32).at[0, :M + 6].set(b)
    memt = mem.T

    grid = (B // BT,)
    r, w = pl.pallas_call(
        _read_head_body,
        grid=grid,
        in_specs=[
            pl.BlockSpec((B, C), lambda i: (0, 0)),
            pl.BlockSpec((BT, N), lambda i: (i, 0)),
            pl.BlockSpec((WPAD, C), lambda i: (0, 0)),
            pl.BlockSpec((1, WPAD), lambda i: (0, 0)),
            pl.BlockSpec((M, N), lambda i: (0, 0)),
        ],
        out_specs=[
            pl.BlockSpec((BT, M), lambda i: (i, 0)),
            pl.BlockSpec((BT, N), lambda i: (i, 0)),
        ],
        out_shape=[
            jax.ShapeDtypeStruct((B, M), jnp.float32),
            jax.ShapeDtypeStruct((B, N), jnp.float32),
        ],
        scratch_shapes=[pltpu.VMEM((M, N), jnp.float32),
                        pltpu.VMEM((B, WPAD), jnp.float32)],
        compiler_params=pltpu.CompilerParams(
            dimension_semantics=("arbitrary",),
        ),
    )(inputs, w_pre, Wp, bp, memt)
    return (r, w)


# speedup vs baseline: 1.0813x; 1.0109x over previous
import jax
import jax.numpy as jnp
from jax.experimental import pallas as pl
from jax.experimental.pallas import tpu as pltpu

B, C, N, M = 1024, 1024, 16384, 64
BT = 64
WPAD = 128


def _read_head_body(x_ref, wpre_ref, W_ref, b_ref, memt_ref, r_ref, w_ref,
                    mnt_ref, fc_ref):
    memt = memt_ref[...]
    i = pl.program_id(0)

    @pl.when(i == 0)
    def _():
        mnorm = jnp.sqrt(jnp.sum(memt * memt, axis=0, keepdims=True))
        mnt_ref[...] = memt * (1.0 / (mnorm + 1e-8))
        fc_ref[...] = jax.lax.dot_general(
            x_ref[...], W_ref[...], (((1,), (1,)), ((), ())),
            preferred_element_type=jnp.float32) + b_ref[...]

    out = fc_ref[pl.ds(i * BT, BT), :]

    k = out[:, :M]
    beta = jax.nn.softplus(out[:, M:M + 1])
    g = jax.nn.sigmoid(out[:, M + 1:M + 2])
    s = jax.nn.softmax(out[:, M + 2:M + 5], axis=1)
    gamma = 1.0 + jax.nn.softplus(out[:, M + 5:M + 6])

    knorm = jnp.sqrt(jnp.sum(k * k, axis=1, keepdims=True))
    kn = k / (knorm + 1e-8)
    sim = jax.lax.dot_general(
        kn, mnt_ref[...], (((1,), (0,)), ((), ())),
        preferred_element_type=jnp.float32)

    e = jnp.exp2((beta * 1.4426950408889634) * sim)
    c1 = g / jnp.sum(e, axis=1, keepdims=True)
    wg = c1 * e + (1.0 - g) * wpre_ref[...]

    ws = (s[:, 0:1] * pltpu.roll(wg, N - 1, 1)
          + s[:, 1:2] * wg
          + s[:, 2:3] * pltpu.roll(wg, 1, 1))

    wp = jnp.exp2(gamma * jnp.log2(ws + 1e-12))
    w = wp * (1.0 / (jnp.sum(wp, axis=1, keepdims=True) + 1e-12))

    w_ref[...] = w
    r_ref[...] = jax.lax.dot_general(
        w, memt, (((1,), (1,)), ((), ())),
        preferred_element_type=jnp.float32)


@jax.jit
def kernel(inputs, w_pre, W, b, mem):
    Wp = jnp.zeros((WPAD, C), jnp.float32).at[:M + 6, :].set(W)
    bp = jnp.zeros((1, WPAD), jnp.float32).at[0, :M + 6].set(b)
    memt = mem.T

    grid = (B // BT,)
    r, w = pl.pallas_call(
        _read_head_body,
        grid=grid,
        in_specs=[
            pl.BlockSpec((B, C), lambda i: (0, 0)),
            pl.BlockSpec((BT, N), lambda i: (i, 0)),
            pl.BlockSpec((WPAD, C), lambda i: (0, 0)),
            pl.BlockSpec((1, WPAD), lambda i: (0, 0)),
            pl.BlockSpec((M, N), lambda i: (0, 0)),
        ],
        out_specs=[
            pl.BlockSpec((BT, M), lambda i: (i, 0)),
            pl.BlockSpec((BT, N), lambda i: (i, 0)),
        ],
        out_shape=[
            jax.ShapeDtypeStruct((B, M), jnp.float32),
            jax.ShapeDtypeStruct((B, N), jnp.float32),
        ],
        scratch_shapes=[pltpu.VMEM((M, N), jnp.float32),
                        pltpu.VMEM((B, WPAD), jnp.float32)],
        compiler_params=pltpu.CompilerParams(
            dimension_semantics=("arbitrary",),
        ),
    )(inputs, w_pre, Wp, bp, memt)
    return (r, w)
